# baseline (device time: 394548 ns/iter reference)
import functools

import jax
import jax.numpy as jnp
from jax import lax
from jax.experimental import pallas as pl
from jax.experimental.pallas import tpu as pltpu

N_Z = 4
KC = 8

_MESH = pl.DeviceIdType.MESH


def kernel(x):
    m, n = x.shape
    blk = m // N_Z
    chk = blk // KC
    half = blk // 2

    def body(x_hbm, out_ref, pbuf, sbuf, copy_sem,
             precv, psend, srecv, ssend, ag_send, ag_recv):
        my_x = lax.axis_index("x")
        my_y = lax.axis_index("y")
        my_z = lax.axis_index("z")
        q = 2 * my_x + my_y
        row0 = q * blk
        zr = (my_z + 1) % N_Z
        zl = (my_z + N_Z - 1) % N_Z
        is_mid = jnp.logical_or(my_z == 1, my_z == 2)

        r = 2 * my_x + (my_x ^ my_y)
        rn = (r + 1) % 4
        rp = (r + 3) % 4
        gn = rn ^ (rn // 2)
        gp = rp ^ (rp // 2)
        nx, ny = gn // 2, gn % 2
        px, py = gp // 2, gp % 2

        cp = pltpu.make_async_copy(
            x_hbm.at[pl.ds(row0, blk)], out_ref.at[pl.ds(row0, blk)],
            copy_sem,
        )
        cp.start()
        cp.wait()

        barrier_sem = pltpu.get_barrier_semaphore()
        for dev in ((my_x, my_y, zl), (my_x, my_y, zr),
                    (nx, ny, my_z), (px, py, my_z)):
            pl.semaphore_signal(
                barrier_sem, inc=1, device_id=dev, device_id_type=_MESH,
            )
        pl.semaphore_wait(barrier_sem, 4)

        def pchunk(ref, c):
            return ref.at[pl.ds(c * chk, chk)]

        def ochunk(c):
            return out_ref.at[pl.ds(row0 + c * chk, chk)]

        S2, S1, S0 = 2, 4, 6
        LAGA = S0 + 1
        for c in range(KC + LAGA + 3):
            if c < KC:
                @pl.when(my_z == 0)
                def _():
                    pltpu.make_async_remote_copy(
                        src_ref=ochunk(c), dst_ref=pchunk(pbuf, c),
                        send_sem=psend.at[c], recv_sem=precv.at[c],
                        device_id=(my_x, my_y, zr), device_id_type=_MESH,
                    ).start()

                @pl.when(my_z == 3)
                def _():
                    pltpu.make_async_remote_copy(
                        src_ref=ochunk(c), dst_ref=pchunk(sbuf, c),
                        send_sem=ssend.at[c], recv_sem=srecv.at[c],
                        device_id=(my_x, my_y, zl), device_id_type=_MESH,
                    ).start()

                @pl.when(is_mid)
                def _():
                    pltpu.make_async_remote_copy(
                        src_ref=pchunk(pbuf, c), dst_ref=pchunk(pbuf, c),
                        send_sem=psend.at[c], recv_sem=precv.at[c],
                        device_id=(my_x, my_y, zl), device_id_type=_MESH,
                    ).wait_recv()
                    pbuf[pl.ds(c * chk, chk), :] = (
                        pbuf[pl.ds(c * chk, chk), :]
                        + out_ref[pl.ds(row0 + c * chk, chk), :]
                    )
                    pltpu.make_async_remote_copy(
                        src_ref=pchunk(pbuf, c), dst_ref=pchunk(pbuf, c),
                        send_sem=psend.at[c], recv_sem=precv.at[c],
                        device_id=(my_x, my_y, zr), device_id_type=_MESH,
                    ).start()

                @pl.when(my_z == 3)
                def _():
                    pltpu.make_async_remote_copy(
                        src_ref=pchunk(pbuf, c), dst_ref=pchunk(pbuf, c),
                        send_sem=psend.at[c], recv_sem=precv.at[c],
                        device_id=(my_x, my_y, zl), device_id_type=_MESH,
                    ).wait_recv()
                    pbuf[pl.ds(c * chk, chk), :] = (
                        pbuf[pl.ds(c * chk, chk), :]
                        + out_ref[pl.ds(row0 + c * chk, chk), :]
                    )
                    pltpu.make_async_remote_copy(
                        src_ref=ochunk(c), dst_ref=pchunk(sbuf, c),
                        send_sem=ssend.at[c], recv_sem=srecv.at[c],
                        device_id=(my_x, my_y, zl), device_id_type=_MESH,
                    ).wait_send()
                    out_ref[pl.ds(row0 + c * chk, chk), :] = pbuf[
                        pl.ds(c * chk, chk), :
                    ]

            c2 = c - S2
            if 0 <= c2 < KC:
                @pl.when(my_z == 2)
                def _():
                    pltpu.make_async_remote_copy(
                        src_ref=pchunk(sbuf, c2), dst_ref=pchunk(sbuf, c2),
                        send_sem=ssend.at[c2], recv_sem=srecv.at[c2],
                        device_id=(my_x, my_y, zr), device_id_type=_MESH,
                    ).wait_recv()
                    sbuf[pl.ds(c2 * chk, chk), :] = (
                        sbuf[pl.ds(c2 * chk, chk), :]
                        + out_ref[pl.ds(row0 + c2 * chk, chk), :]
                    )
                    pltpu.make_async_remote_copy(
                        src_ref=pchunk(sbuf, c2), dst_ref=pchunk(sbuf, c2),
                        send_sem=ssend.at[c2], recv_sem=srecv.at[c2],
                        device_id=(my_x, my_y, zl), device_id_type=_MESH,
                    ).start()
                    out_ref[pl.ds(row0 + c2 * chk, chk), :] = (
                        pbuf[pl.ds(c2 * chk, chk), :]
                        + sbuf[pl.ds(c2 * chk, chk), :]
                        - out_ref[pl.ds(row0 + c2 * chk, chk), :]
                    )

            c1 = c - S1
            if 0 <= c1 < KC:
                @pl.when(my_z == 1)
                def _():
                    pltpu.make_async_remote_copy(
                        src_ref=pchunk(sbuf, c1), dst_ref=pchunk(sbuf, c1),
                        send_sem=ssend.at[c1], recv_sem=srecv.at[c1],
                        device_id=(my_x, my_y, zr), device_id_type=_MESH,
                    ).wait_recv()
                    sbuf[pl.ds(c1 * chk, chk), :] = (
                        sbuf[pl.ds(c1 * chk, chk), :]
                        + out_ref[pl.ds(row0 + c1 * chk, chk), :]
                    )
                    pltpu.make_async_remote_copy(
                        src_ref=pchunk(sbuf, c1), dst_ref=pchunk(sbuf, c1),
                        send_sem=ssend.at[c1], recv_sem=srecv.at[c1],
                        device_id=(my_x, my_y, zl), device_id_type=_MESH,
                    ).start()
                    out_ref[pl.ds(row0 + c1 * chk, chk), :] = (
                        pbuf[pl.ds(c1 * chk, chk), :]
                        + sbuf[pl.ds(c1 * chk, chk), :]
                        - out_ref[pl.ds(row0 + c1 * chk, chk), :]
                    )

            c0 = c - S0
            if 0 <= c0 < KC:
                @pl.when(my_z == 0)
                def _():
                    pltpu.make_async_remote_copy(
                        src_ref=pchunk(sbuf, c0), dst_ref=pchunk(sbuf, c0),
                        send_sem=ssend.at[c0], recv_sem=srecv.at[c0],
                        device_id=(my_x, my_y, zr), device_id_type=_MESH,
                    ).wait_recv()
                    pltpu.make_async_remote_copy(
                        src_ref=ochunk(c0), dst_ref=pchunk(pbuf, c0),
                        send_sem=psend.at[c0], recv_sem=precv.at[c0],
                        device_id=(my_x, my_y, zr), device_id_type=_MESH,
                    ).wait_send()
                    out_ref[pl.ds(row0 + c0 * chk, chk), :] = (
                        out_ref[pl.ds(row0 + c0 * chk, chk), :]
                        + sbuf[pl.ds(c0 * chk, chk), :]
                    )

            for t in range(3):
                j = c - LAGA - t
                if 0 <= j < KC:
                    if j < KC // 2:
                        rr = (r + 4 - t) % 4
                        dev = (nx, ny, my_z)
                    else:
                        rr = (r + t) % 4
                        dev = (px, py, my_z)
                    b = rr ^ (rr // 2)
                    rows = b * blk + j * chk
                    ag = pltpu.make_async_remote_copy(
                        src_ref=out_ref.at[pl.ds(rows, chk)],
                        dst_ref=out_ref.at[pl.ds(rows, chk)],
                        send_sem=ag_send.at[j * 3 + t],
                        recv_sem=ag_recv.at[j * 3 + t],
                        device_id=dev, device_id_type=_MESH,
                    )
                    if t > 0:
                        pltpu.make_async_remote_copy(
                            src_ref=out_ref.at[pl.ds(rows, chk)],
                            dst_ref=out_ref.at[pl.ds(rows, chk)],
                            send_sem=ag_send.at[j * 3 + t - 1],
                            recv_sem=ag_recv.at[j * 3 + t - 1],
                            device_id=dev, device_id_type=_MESH,
                        ).wait_recv()
                    ag.start()

            jf = c - LAGA - 3
            if 0 <= jf < KC:
                if jf < KC // 2:
                    rrf = (r + 1) % 4
                else:
                    rrf = (r + 3) % 4
                bf = rrf ^ (rrf // 2)
                rowsf = bf * blk + jf * chk
                pltpu.make_async_remote_copy(
                    src_ref=out_ref.at[pl.ds(rowsf, chk)],
                    dst_ref=out_ref.at[pl.ds(rowsf, chk)],
                    send_sem=ag_send.at[jf * 3 + 2],
                    recv_sem=ag_recv.at[jf * 3 + 2],
                    device_id=(nx, ny, my_z), device_id_type=_MESH,
                ).wait_recv()

        @pl.when(is_mid)
        def _():
            for c in range(KC):
                pltpu.make_async_remote_copy(
                    src_ref=pchunk(pbuf, c), dst_ref=pchunk(pbuf, c),
                    send_sem=psend.at[c], recv_sem=precv.at[c],
                    device_id=(my_x, my_y, zr), device_id_type=_MESH,
                ).wait_send()
                pltpu.make_async_remote_copy(
                    src_ref=pchunk(sbuf, c), dst_ref=pchunk(sbuf, c),
                    send_sem=ssend.at[c], recv_sem=srecv.at[c],
                    device_id=(my_x, my_y, zl), device_id_type=_MESH,
                ).wait_send()

        for j in range(KC):
            for t in range(3):
                pltpu.make_async_remote_copy(
                    src_ref=out_ref.at[pl.ds(j * chk, chk)],
                    dst_ref=out_ref.at[pl.ds(j * chk, chk)],
                    send_sem=ag_send.at[j * 3 + t],
                    recv_sem=ag_recv.at[j * 3 + t],
                    device_id=(nx, ny, my_z), device_id_type=_MESH,
                ).wait_send()

        @functools.partial(
            pl.run_scoped, exit_sem=pltpu.SemaphoreType.REGULAR
        )
        def _(exit_sem):
            for dev in ((my_x, my_y, zl), (my_x, my_y, zr),
                        (nx, ny, my_z), (px, py, my_z)):
                pl.semaphore_signal(
                    exit_sem, inc=1, device_id=dev, device_id_type=_MESH,
                )
            pl.semaphore_wait(exit_sem, 4)

    return pl.pallas_call(
        body,
        out_shape=jax.ShapeDtypeStruct((m, n), x.dtype),
        in_specs=[pl.BlockSpec(memory_space=pl.ANY)],
        out_specs=pl.BlockSpec(memory_space=pltpu.VMEM),
        scratch_shapes=[
            pltpu.VMEM((blk, n), x.dtype),
            pltpu.VMEM((blk, n), x.dtype),
            pltpu.SemaphoreType.DMA,
            pltpu.SemaphoreType.DMA((KC,)),
            pltpu.SemaphoreType.DMA((KC,)),
            pltpu.SemaphoreType.DMA((KC,)),
            pltpu.SemaphoreType.DMA((KC,)),
            pltpu.SemaphoreType.DMA((KC * 3,)),
            pltpu.SemaphoreType.DMA((KC * 3,)),
        ],
        compiler_params=pltpu.CompilerParams(
            collective_id=0,
            vmem_limit_bytes=100 * 1024 * 1024,
        ),
    )(x)


# device time: 327130 ns/iter; 1.2061x vs baseline; 1.2061x over previous
import functools

import jax
import jax.numpy as jnp
from jax import lax
from jax.experimental import pallas as pl
from jax.experimental.pallas import tpu as pltpu

N_Z = 4
KC = 8

_MESH = pl.DeviceIdType.MESH


def kernel(x):
    m, n = x.shape
    blk = m // N_Z
    chk = blk // KC
    half = blk // 2

    def body(x_hbm, out_ref, pbuf, sbuf, copy_sem,
             precv, psend, srecv, ssend, ag_send, ag_recv):
        my_x = lax.axis_index("x")
        my_y = lax.axis_index("y")
        my_z = lax.axis_index("z")
        q = 2 * my_x + my_y
        row0 = q * blk
        zr = (my_z + 1) % N_Z
        zl = (my_z + N_Z - 1) % N_Z
        is_mid = jnp.logical_or(my_z == 1, my_z == 2)

        r = 2 * my_x + (my_x ^ my_y)
        rn = (r + 1) % 4
        rp = (r + 3) % 4
        gn = rn ^ (rn // 2)
        gp = rp ^ (rp // 2)
        nx, ny = gn // 2, gn % 2
        px, py = gp // 2, gp % 2

        cp = pltpu.make_async_copy(
            x_hbm.at[pl.ds(row0, blk)], out_ref.at[pl.ds(row0, blk)],
            copy_sem,
        )
        cp.start()
        cp.wait()

        barrier_sem = pltpu.get_barrier_semaphore()
        for dev in ((my_x, my_y, zl), (my_x, my_y, zr),
                    (nx, ny, my_z), (px, py, my_z)):
            pl.semaphore_signal(
                barrier_sem, inc=1, device_id=dev, device_id_type=_MESH,
            )
        pl.semaphore_wait(barrier_sem, 4)

        def pchunk(ref, c):
            return ref.at[pl.ds(c * chk, chk)]

        def ochunk(c):
            return out_ref.at[pl.ds(row0 + c * chk, chk)]

        S2, S1, S0 = 2, 4, 6
        LAGA = S0 + 1
        for c in range(KC + LAGA + 10):
            if c < KC:
                @pl.when(my_z == 0)
                def _():
                    pltpu.make_async_remote_copy(
                        src_ref=ochunk(c), dst_ref=pchunk(pbuf, c),
                        send_sem=psend.at[c], recv_sem=precv.at[c],
                        device_id=(my_x, my_y, zr), device_id_type=_MESH,
                    ).start()

                @pl.when(my_z == 3)
                def _():
                    pltpu.make_async_remote_copy(
                        src_ref=ochunk(c), dst_ref=pchunk(sbuf, c),
                        send_sem=ssend.at[c], recv_sem=srecv.at[c],
                        device_id=(my_x, my_y, zl), device_id_type=_MESH,
                    ).start()

                @pl.when(is_mid)
                def _():
                    pltpu.make_async_remote_copy(
                        src_ref=pchunk(pbuf, c), dst_ref=pchunk(pbuf, c),
                        send_sem=psend.at[c], recv_sem=precv.at[c],
                        device_id=(my_x, my_y, zl), device_id_type=_MESH,
                    ).wait_recv()
                    pbuf[pl.ds(c * chk, chk), :] = (
                        pbuf[pl.ds(c * chk, chk), :]
                        + out_ref[pl.ds(row0 + c * chk, chk), :]
                    )
                    pltpu.make_async_remote_copy(
                        src_ref=pchunk(pbuf, c), dst_ref=pchunk(pbuf, c),
                        send_sem=psend.at[c], recv_sem=precv.at[c],
                        device_id=(my_x, my_y, zr), device_id_type=_MESH,
                    ).start()

                @pl.when(my_z == 3)
                def _():
                    pltpu.make_async_remote_copy(
                        src_ref=pchunk(pbuf, c), dst_ref=pchunk(pbuf, c),
                        send_sem=psend.at[c], recv_sem=precv.at[c],
                        device_id=(my_x, my_y, zl), device_id_type=_MESH,
                    ).wait_recv()
                    pbuf[pl.ds(c * chk, chk), :] = (
                        pbuf[pl.ds(c * chk, chk), :]
                        + out_ref[pl.ds(row0 + c * chk, chk), :]
                    )
                    pltpu.make_async_remote_copy(
                        src_ref=ochunk(c), dst_ref=pchunk(sbuf, c),
                        send_sem=ssend.at[c], recv_sem=srecv.at[c],
                        device_id=(my_x, my_y, zl), device_id_type=_MESH,
                    ).wait_send()
                    out_ref[pl.ds(row0 + c * chk, chk), :] = pbuf[
                        pl.ds(c * chk, chk), :
                    ]

            c2 = c - S2
            if 0 <= c2 < KC:
                @pl.when(my_z == 2)
                def _():
                    pltpu.make_async_remote_copy(
                        src_ref=pchunk(sbuf, c2), dst_ref=pchunk(sbuf, c2),
                        send_sem=ssend.at[c2], recv_sem=srecv.at[c2],
                        device_id=(my_x, my_y, zr), device_id_type=_MESH,
                    ).wait_recv()
                    sbuf[pl.ds(c2 * chk, chk), :] = (
                        sbuf[pl.ds(c2 * chk, chk), :]
                        + out_ref[pl.ds(row0 + c2 * chk, chk), :]
                    )
                    pltpu.make_async_remote_copy(
                        src_ref=pchunk(sbuf, c2), dst_ref=pchunk(sbuf, c2),
                        send_sem=ssend.at[c2], recv_sem=srecv.at[c2],
                        device_id=(my_x, my_y, zl), device_id_type=_MESH,
                    ).start()
                    out_ref[pl.ds(row0 + c2 * chk, chk), :] = (
                        pbuf[pl.ds(c2 * chk, chk), :]
                        + sbuf[pl.ds(c2 * chk, chk), :]
                        - out_ref[pl.ds(row0 + c2 * chk, chk), :]
                    )

            c1 = c - S1
            if 0 <= c1 < KC:
                @pl.when(my_z == 1)
                def _():
                    pltpu.make_async_remote_copy(
                        src_ref=pchunk(sbuf, c1), dst_ref=pchunk(sbuf, c1),
                        send_sem=ssend.at[c1], recv_sem=srecv.at[c1],
                        device_id=(my_x, my_y, zr), device_id_type=_MESH,
                    ).wait_recv()
                    sbuf[pl.ds(c1 * chk, chk), :] = (
                        sbuf[pl.ds(c1 * chk, chk), :]
                        + out_ref[pl.ds(row0 + c1 * chk, chk), :]
                    )
                    pltpu.make_async_remote_copy(
                        src_ref=pchunk(sbuf, c1), dst_ref=pchunk(sbuf, c1),
                        send_sem=ssend.at[c1], recv_sem=srecv.at[c1],
                        device_id=(my_x, my_y, zl), device_id_type=_MESH,
                    ).start()
                    out_ref[pl.ds(row0 + c1 * chk, chk), :] = (
                        pbuf[pl.ds(c1 * chk, chk), :]
                        + sbuf[pl.ds(c1 * chk, chk), :]
                        - out_ref[pl.ds(row0 + c1 * chk, chk), :]
                    )

            c0 = c - S0
            if 0 <= c0 < KC:
                @pl.when(my_z == 0)
                def _():
                    pltpu.make_async_remote_copy(
                        src_ref=pchunk(sbuf, c0), dst_ref=pchunk(sbuf, c0),
                        send_sem=ssend.at[c0], recv_sem=srecv.at[c0],
                        device_id=(my_x, my_y, zr), device_id_type=_MESH,
                    ).wait_recv()
                    pltpu.make_async_remote_copy(
                        src_ref=ochunk(c0), dst_ref=pchunk(pbuf, c0),
                        send_sem=psend.at[c0], recv_sem=precv.at[c0],
                        device_id=(my_x, my_y, zr), device_id_type=_MESH,
                    ).wait_send()
                    out_ref[pl.ds(row0 + c0 * chk, chk), :] = (
                        out_ref[pl.ds(row0 + c0 * chk, chk), :]
                        + sbuf[pl.ds(c0 * chk, chk), :]
                    )

            for t in range(3):
                j = c - LAGA - 3 * t
                if 0 <= j < KC:
                    if j < KC // 2:
                        rr = (r + 4 - t) % 4
                        dev = (nx, ny, my_z)
                    else:
                        rr = (r + t) % 4
                        dev = (px, py, my_z)
                    b = rr ^ (rr // 2)
                    rows = b * blk + j * chk
                    ag = pltpu.make_async_remote_copy(
                        src_ref=out_ref.at[pl.ds(rows, chk)],
                        dst_ref=out_ref.at[pl.ds(rows, chk)],
                        send_sem=ag_send.at[j * 3 + t],
                        recv_sem=ag_recv.at[j * 3 + t],
                        device_id=dev, device_id_type=_MESH,
                    )
                    if t > 0:
                        pltpu.make_async_remote_copy(
                            src_ref=out_ref.at[pl.ds(rows, chk)],
                            dst_ref=out_ref.at[pl.ds(rows, chk)],
                            send_sem=ag_send.at[j * 3 + t - 1],
                            recv_sem=ag_recv.at[j * 3 + t - 1],
                            device_id=dev, device_id_type=_MESH,
                        ).wait_recv()
                    ag.start()

            jf = c - LAGA - 9
            if 0 <= jf < KC:
                if jf < KC // 2:
                    rrf = (r + 1) % 4
                else:
                    rrf = (r + 3) % 4
                bf = rrf ^ (rrf // 2)
                rowsf = bf * blk + jf * chk
                pltpu.make_async_remote_copy(
                    src_ref=out_ref.at[pl.ds(rowsf, chk)],
                    dst_ref=out_ref.at[pl.ds(rowsf, chk)],
                    send_sem=ag_send.at[jf * 3 + 2],
                    recv_sem=ag_recv.at[jf * 3 + 2],
                    device_id=(nx, ny, my_z), device_id_type=_MESH,
                ).wait_recv()

        @pl.when(is_mid)
        def _():
            for c in range(KC):
                pltpu.make_async_remote_copy(
                    src_ref=pchunk(pbuf, c), dst_ref=pchunk(pbuf, c),
                    send_sem=psend.at[c], recv_sem=precv.at[c],
                    device_id=(my_x, my_y, zr), device_id_type=_MESH,
                ).wait_send()
                pltpu.make_async_remote_copy(
                    src_ref=pchunk(sbuf, c), dst_ref=pchunk(sbuf, c),
                    send_sem=ssend.at[c], recv_sem=srecv.at[c],
                    device_id=(my_x, my_y, zl), device_id_type=_MESH,
                ).wait_send()

        for j in range(KC):
            for t in range(3):
                pltpu.make_async_remote_copy(
                    src_ref=out_ref.at[pl.ds(j * chk, chk)],
                    dst_ref=out_ref.at[pl.ds(j * chk, chk)],
                    send_sem=ag_send.at[j * 3 + t],
                    recv_sem=ag_recv.at[j * 3 + t],
                    device_id=(nx, ny, my_z), device_id_type=_MESH,
                ).wait_send()

        @functools.partial(
            pl.run_scoped, exit_sem=pltpu.SemaphoreType.REGULAR
        )
        def _(exit_sem):
            for dev in ((my_x, my_y, zl), (my_x, my_y, zr),
                        (nx, ny, my_z), (px, py, my_z)):
                pl.semaphore_signal(
                    exit_sem, inc=1, device_id=dev, device_id_type=_MESH,
                )
            pl.semaphore_wait(exit_sem, 4)

    return pl.pallas_call(
        body,
        out_shape=jax.ShapeDtypeStruct((m, n), x.dtype),
        in_specs=[pl.BlockSpec(memory_space=pl.ANY)],
        out_specs=pl.BlockSpec(memory_space=pltpu.VMEM),
        scratch_shapes=[
            pltpu.VMEM((blk, n), x.dtype),
            pltpu.VMEM((blk, n), x.dtype),
            pltpu.SemaphoreType.DMA,
            pltpu.SemaphoreType.DMA((KC,)),
            pltpu.SemaphoreType.DMA((KC,)),
            pltpu.SemaphoreType.DMA((KC,)),
            pltpu.SemaphoreType.DMA((KC,)),
            pltpu.SemaphoreType.DMA((KC * 3,)),
            pltpu.SemaphoreType.DMA((KC * 3,)),
        ],
        compiler_params=pltpu.CompilerParams(
            collective_id=0,
            vmem_limit_bytes=100 * 1024 * 1024,
        ),
    )(x)
